# Initial kernel scaffold; baseline (speedup 1.0000x reference)
import jax
import jax.numpy as jnp
from jax.experimental import pallas as pl


def kernel(time, table):
    raise NotImplementedError("write your pallas kernel here")


# speedup vs baseline: 2.2882x; 2.2882x over previous
import functools

import jax
import jax.numpy as jnp
from jax import lax
from jax.experimental import pallas as pl
from jax.experimental.pallas import tpu as pltpu
from jax.experimental.pallas import tpu_sc as plsc

NUM_EMB = 49
EMBED = 64
NC = 2
NS = 16
NW = NC * NS

CHUNK = 128
BLOCK = 8


@functools.partial(jax.jit, static_argnames=("b_tot",))
def _sc_embedding_lookup(idx2d, table, *, b_tot):
    rows_tot = b_tot // CHUNK
    rows_per_w = rows_tot // NW
    n_outer = rows_per_w // BLOCK

    mesh = plsc.VectorSubcoreMesh(core_axis_name="c", subcore_axis_name="s")

    @functools.partial(
        pl.kernel,
        mesh=mesh,
        compiler_params=pltpu.CompilerParams(use_tc_tiling_on_sc=False),
        out_type=jax.ShapeDtypeStruct((b_tot, EMBED), jnp.float32),
        scratch_types=dict(
            idx_v=pltpu.VMEM((BLOCK, CHUNK), jnp.int32),
            rows_v=pltpu.VMEM((BLOCK, CHUNK, EMBED), jnp.float32),
            sem_g=pltpu.SemaphoreType.DMA,
            sem_w=pltpu.SemaphoreType.DMA,
        ),
    )
    def k(idx_hbm, table_hbm, out_hbm, idx_v, rows_v, sem_g, sem_w):
        wid = lax.axis_index("s") * NC + lax.axis_index("c")
        base_row = wid * rows_per_w

        def outer(i, _):
            row0 = base_row + i * BLOCK
            pltpu.sync_copy(idx_hbm.at[pl.ds(row0, BLOCK), :], idx_v)
            gathers = []
            for j in range(BLOCK):
                gathers.append(
                    pltpu.async_copy(
                        table_hbm.at[idx_v.at[j]], rows_v.at[j], sem_g
                    )
                )
            writes = []
            for j in range(BLOCK):
                gathers[j].wait()
                writes.append(
                    pltpu.async_copy(
                        rows_v.at[j],
                        out_hbm.at[pl.ds((row0 + j) * CHUNK, CHUNK)],
                        sem_w,
                    )
                )
            for j in range(BLOCK):
                writes[j].wait()
            return ()

        lax.fori_loop(0, n_outer, outer, (), unroll=False)

    return k(idx2d, table)


def kernel(time, table):
    b, h = time.shape
    idx2d = time.reshape(b * h // CHUNK, CHUNK).astype(jnp.int32)
    out = _sc_embedding_lookup(idx2d, table, b_tot=b * h)
    return out.reshape(b, h, EMBED)


# speedup vs baseline: 2.2905x; 1.0010x over previous
import functools

import jax
import jax.numpy as jnp
from jax import lax
from jax.experimental import pallas as pl
from jax.experimental.pallas import tpu as pltpu
from jax.experimental.pallas import tpu_sc as plsc

NUM_EMB = 49
EMBED = 64
NC = 2
NS = 16
NW = NC * NS

CHUNK = 128
BLOCK = 5


@functools.partial(jax.jit, static_argnames=("b_tot",))
def _sc_embedding_lookup(idx2d, table, *, b_tot):
    rows_tot = b_tot // CHUNK
    rows_per_w = rows_tot // NW
    n_blocks = rows_per_w // BLOCK
    n_outer = n_blocks // 2

    mesh = plsc.VectorSubcoreMesh(core_axis_name="c", subcore_axis_name="s")

    @functools.partial(
        pl.kernel,
        mesh=mesh,
        compiler_params=pltpu.CompilerParams(use_tc_tiling_on_sc=False),
        out_type=jax.ShapeDtypeStruct((b_tot, EMBED), jnp.float32),
        scratch_types=dict(
            idx_v=pltpu.VMEM((2, BLOCK, CHUNK), jnp.int32),
            rows_v=pltpu.VMEM((2, BLOCK, CHUNK, EMBED), jnp.float32),
            sem_i=pltpu.SemaphoreType.DMA,
            sem_g=pltpu.SemaphoreType.DMA,
            sem_w=pltpu.SemaphoreType.DMA,
        ),
    )
    def k(idx_hbm, table_hbm, out_hbm, idx_v, rows_v, sem_i, sem_g, sem_w):
        wid = lax.axis_index("s") * NC + lax.axis_index("c")
        base_row = wid * rows_per_w

        def load_idx(blk, slot):
            row0 = base_row + blk * BLOCK
            pltpu.async_copy(
                idx_hbm.at[pl.ds(row0, BLOCK), :], idx_v.at[slot], sem_i
            )

        def drain_idx(slot):
            pltpu.make_async_copy(
                idx_hbm.at[pl.ds(base_row, BLOCK), :], idx_v.at[slot], sem_i
            ).wait()

        def fire_gathers(slot):
            for j in range(BLOCK):
                pltpu.async_copy(
                    table_hbm.at[idx_v.at[slot, j]], rows_v.at[slot, j], sem_g
                )

        def fire_writes(blk, slot):
            row0 = base_row + blk * BLOCK
            for j in range(BLOCK):
                pltpu.make_async_copy(
                    table_hbm.at[idx_v.at[slot, j]], rows_v.at[slot, j], sem_g
                ).wait()
                pltpu.async_copy(
                    rows_v.at[slot, j],
                    out_hbm.at[pl.ds((row0 + j) * CHUNK, CHUNK)],
                    sem_w,
                )

        def drain_writes(blk, slot):
            row0 = base_row + blk * BLOCK
            for j in range(BLOCK):
                pltpu.make_async_copy(
                    rows_v.at[slot, j],
                    out_hbm.at[pl.ds((row0 + j) * CHUNK, CHUNK)],
                    sem_w,
                ).wait()

        def step(blk, slot, prefetch):
            other = 1 - slot
            fire_writes(blk, slot)
            drain_idx(other)
            if prefetch:
                load_idx(blk + 2, slot)
            fire_gathers(other)
            drain_writes(blk, slot)

        load_idx(0, 0)
        drain_idx(0)
        load_idx(1, 1)
        fire_gathers(0)

        def outer(i, carry):
            blk = i * 2
            step(blk, 0, True)
            step(blk + 1, 1, True)
            return carry

        lax.fori_loop(0, n_outer - 1, outer, 0, unroll=False)

        blk = (n_outer - 1) * 2
        step(blk, 0, False)
        fire_writes(blk + 1, 1)
        drain_writes(blk + 1, 1)

    return k(idx2d, table)


def kernel(time, table):
    b, h = time.shape
    idx2d = time.reshape(b * h // CHUNK, CHUNK).astype(jnp.int32)
    out = _sc_embedding_lookup(idx2d, table, b_tot=b * h)
    return out.reshape(b, h, EMBED)


# speedup vs baseline: 5.8078x; 2.5356x over previous
import functools

import jax
import jax.numpy as jnp
from jax import lax
from jax.experimental import pallas as pl
from jax.experimental.pallas import tpu as pltpu
from jax.experimental.pallas import tpu_sc as plsc

NUM_EMB = 49
EMBED = 64
NC = 2
NS = 16
NW = NC * NS

CHUNK = 128
BLOCK = 5


@functools.partial(jax.jit, static_argnames=("b_tot",))
def _sc_embedding_lookup(idx2d, table, *, b_tot):
    rows_tot = b_tot // CHUNK
    rows_per_w = rows_tot // NW
    n_blocks = rows_per_w // BLOCK
    n_outer = n_blocks // 2

    mesh = plsc.VectorSubcoreMesh(core_axis_name="c", subcore_axis_name="s")

    @functools.partial(
        pl.kernel,
        mesh=mesh,
        compiler_params=pltpu.CompilerParams(use_tc_tiling_on_sc=False),
        out_type=jax.ShapeDtypeStruct((b_tot, EMBED), jnp.float32),
        scratch_types=dict(
            idx_v=pltpu.VMEM((2, BLOCK, CHUNK), jnp.int32),
            rows_v=pltpu.VMEM((2, BLOCK, CHUNK, EMBED), jnp.float32),
            table_v=pltpu.VMEM_SHARED((NUM_EMB, EMBED), jnp.float32),
            sem_i=pltpu.SemaphoreType.DMA,
            sem_g=pltpu.SemaphoreType.DMA,
            sem_w=pltpu.SemaphoreType.DMA,
        ),
    )
    def k(idx_hbm, table_hbm, out_hbm, idx_v, rows_v, table_v,
          sem_i, sem_g, sem_w):
        wid = lax.axis_index("s") * NC + lax.axis_index("c")
        base_row = wid * rows_per_w
        @pl.when(lax.axis_index("s") == 0)
        def _():
            pltpu.sync_copy(table_hbm, table_v)
        plsc.subcore_barrier()

        def load_idx(blk, slot):
            row0 = base_row + blk * BLOCK
            pltpu.async_copy(
                idx_hbm.at[pl.ds(row0, BLOCK), :], idx_v.at[slot], sem_i
            )

        def drain_idx(slot):
            pltpu.make_async_copy(
                idx_hbm.at[pl.ds(base_row, BLOCK), :], idx_v.at[slot], sem_i
            ).wait()

        def fire_gathers(slot):
            for j in range(BLOCK):
                pltpu.async_copy(
                    table_v.at[idx_v.at[slot, j]], rows_v.at[slot, j], sem_g
                )

        def fire_writes(blk, slot):
            row0 = base_row + blk * BLOCK
            for j in range(BLOCK):
                pltpu.make_async_copy(
                    table_v.at[idx_v.at[slot, j]], rows_v.at[slot, j], sem_g
                ).wait()
                pltpu.async_copy(
                    rows_v.at[slot, j],
                    out_hbm.at[pl.ds((row0 + j) * CHUNK, CHUNK)],
                    sem_w,
                )

        def drain_writes(blk, slot):
            row0 = base_row + blk * BLOCK
            for j in range(BLOCK):
                pltpu.make_async_copy(
                    rows_v.at[slot, j],
                    out_hbm.at[pl.ds((row0 + j) * CHUNK, CHUNK)],
                    sem_w,
                ).wait()

        def step(blk, slot, prefetch):
            other = 1 - slot
            fire_writes(blk, slot)
            drain_idx(other)
            if prefetch:
                load_idx(blk + 2, slot)
            fire_gathers(other)
            drain_writes(blk, slot)

        load_idx(0, 0)
        drain_idx(0)
        load_idx(1, 1)
        fire_gathers(0)

        def outer(i, carry):
            blk = i * 2
            step(blk, 0, True)
            step(blk + 1, 1, True)
            return carry

        lax.fori_loop(0, n_outer - 1, outer, 0, unroll=False)

        blk = (n_outer - 1) * 2
        step(blk, 0, False)
        fire_writes(blk + 1, 1)
        drain_writes(blk + 1, 1)

    return k(idx2d, table)


def kernel(time, table):
    b, h = time.shape
    idx2d = time.reshape(b * h // CHUNK, CHUNK).astype(jnp.int32)
    out = _sc_embedding_lookup(idx2d, table, b_tot=b * h)
    return out.reshape(b, h, EMBED)
